# baseline (speedup 1.0000x reference)
import functools

import jax
import jax.numpy as jnp
from jax import lax
from jax.experimental import pallas as pl
from jax.experimental.pallas import tpu as pltpu
from jax.experimental.pallas import tpu_sc as plsc

NC = 2
NS = 16
NW = NC * NS
L = 16

D = 64
C = 256


@functools.partial(jax.jit, static_argnames=("n_rows",))
def _fused_lookup(pid, mid, word_table, morph_table, *, n_rows):
    per_w = n_rows // NW
    n_chunks = per_w // C
    assert n_chunks % 2 == 0

    mesh = plsc.VectorSubcoreMesh(core_axis_name="c", subcore_axis_name="s")

    @functools.partial(
        pl.kernel,
        out_type=jax.ShapeDtypeStruct((n_rows, D), jnp.float32),
        mesh=mesh,
        compiler_params=pltpu.CompilerParams(use_tc_tiling_on_sc=False),
        scratch_types=[
            pltpu.VMEM((per_w,), jnp.int32),
            pltpu.VMEM((per_w,), jnp.int32),
            pltpu.VMEM((C, D), jnp.float32),
            pltpu.VMEM((C, D), jnp.float32),
            pltpu.VMEM((C, D), jnp.float32),
            pltpu.VMEM((C, D), jnp.float32),
            pltpu.SemaphoreType.DMA,
            pltpu.SemaphoreType.DMA,
            pltpu.SemaphoreType.DMA,
            pltpu.SemaphoreType.DMA,
        ],
    )
    def body(pid_hbm, mid_hbm, word_hbm, morph_hbm, out_hbm,
             idxw, idxm, roww_a, rowm_a, roww_b, rowm_b,
             semg_a, semg_b, semo_a, semo_b):
        wid = lax.axis_index("s") * NC + lax.axis_index("c")
        base = wid * per_w

        pltpu.sync_copy(pid_hbm.at[pl.ds(base, per_w)], idxw)
        pltpu.sync_copy(mid_hbm.at[pl.ds(base, per_w)], idxm)

        def gather_copies(g, roww, rowm, semg):
            lo = g * C
            cw = pltpu.make_async_copy(
                word_hbm.at[idxw.at[pl.ds(lo, C)]], roww, semg)
            cm = pltpu.make_async_copy(
                morph_hbm.at[idxm.at[pl.ds(lo, C)]], rowm, semg)
            return cw, cm

        def out_copy(g, roww, semo):
            return pltpu.make_async_copy(
                roww, out_hbm.at[pl.ds(base + g * C, C)], semo)

        def step(g, cur, oth):
            roww, rowm, semg, semo = cur
            o_roww, o_rowm, o_semg, o_semo = oth

            @pl.when(g + 1 < n_chunks)
            def _():
                @pl.when(g >= 1)
                def _():
                    out_copy(g - 1, o_roww, o_semo).wait()

                cw, cm = gather_copies(g + 1, o_roww, o_rowm, o_semg)
                cw.start()
                cm.start()

            cw, cm = gather_copies(g, roww, rowm, semg)
            cw.wait()
            cm.wait()

            def row(i, carry):
                for c in range(D // L):
                    a = roww[i, pl.ds(c * L, L)]
                    b = rowm[i, pl.ds(c * L, L)]
                    roww[i, pl.ds(c * L, L)] = (a + b) * 0.5
                return carry

            lax.fori_loop(0, C, row, 0, unroll=2)
            out_copy(g, roww, semo).start()

        buf_a = (roww_a, rowm_a, semg_a, semo_a)
        buf_b = (roww_b, rowm_b, semg_b, semo_b)

        cw, cm = gather_copies(0, roww_a, rowm_a, semg_a)
        cw.start()
        cm.start()

        def super_step(t, carry):
            step(2 * t, buf_a, buf_b)
            step(2 * t + 1, buf_b, buf_a)
            return carry

        lax.fori_loop(0, n_chunks // 2, super_step, 0, unroll=False)
        out_copy(n_chunks - 2, roww_a, semo_a).wait()
        out_copy(n_chunks - 1, roww_b, semo_b).wait()

    return body(pid, mid, word_table, morph_table)


def kernel(phrase_ids, morph_ids, word_table, morph_table):
    batch, seq = phrase_ids.shape
    n_rows = batch * seq
    out = _fused_lookup(
        phrase_ids.reshape(n_rows),
        morph_ids.reshape(n_rows),
        word_table,
        morph_table,
        n_rows=n_rows,
    )
    return out.reshape(batch, seq, D)


# speedup vs baseline: 1.0004x; 1.0004x over previous
import functools

import jax
import jax.numpy as jnp
from jax import lax
from jax.experimental import pallas as pl
from jax.experimental.pallas import tpu as pltpu
from jax.experimental.pallas import tpu_sc as plsc

NC = 2
NS = 16
NW = NC * NS
L = 16
D = 64


def _fused_lookup(pid, mid, word_table, morph_table):
    batch, seq = pid.shape
    rows_w = batch // NW
    assert rows_w % 2 == 0

    mesh = plsc.VectorSubcoreMesh(core_axis_name="c", subcore_axis_name="s")

    @functools.partial(
        pl.kernel,
        out_type=jax.ShapeDtypeStruct((batch, seq, D), jnp.float32),
        mesh=mesh,
        compiler_params=pltpu.CompilerParams(use_tc_tiling_on_sc=False),
        scratch_types=[
            pltpu.VMEM((rows_w, seq), jnp.int32),
            pltpu.VMEM((rows_w, seq), jnp.int32),
            pltpu.VMEM((seq, D), jnp.float32),
            pltpu.VMEM((seq, D), jnp.float32),
            pltpu.VMEM((seq, D), jnp.float32),
            pltpu.VMEM((seq, D), jnp.float32),
            pltpu.SemaphoreType.DMA,
            pltpu.SemaphoreType.DMA,
            pltpu.SemaphoreType.DMA,
            pltpu.SemaphoreType.DMA,
        ],
    )
    def body(pid_hbm, mid_hbm, word_hbm, morph_hbm, out_hbm,
             idxw, idxm, roww_a, rowm_a, roww_b, rowm_b,
             semg_a, semg_b, semo_a, semo_b):
        wid = lax.axis_index("s") * NC + lax.axis_index("c")
        b0 = wid * rows_w

        pltpu.sync_copy(pid_hbm.at[pl.ds(b0, rows_w)], idxw)
        pltpu.sync_copy(mid_hbm.at[pl.ds(b0, rows_w)], idxm)

        def gather_copies(g, roww, rowm, semg):
            cw = pltpu.make_async_copy(word_hbm.at[idxw.at[g]], roww, semg)
            cm = pltpu.make_async_copy(morph_hbm.at[idxm.at[g]], rowm, semg)
            return cw, cm

        def out_copy(g, roww, semo):
            return pltpu.make_async_copy(roww, out_hbm.at[b0 + g], semo)

        def step(g, cur, oth):
            roww, rowm, semg, semo = cur
            o_roww, o_rowm, o_semg, o_semo = oth

            @pl.when(g + 1 < rows_w)
            def _():
                @pl.when(g >= 1)
                def _():
                    out_copy(g - 1, o_roww, o_semo).wait()

                cw, cm = gather_copies(g + 1, o_roww, o_rowm, o_semg)
                cw.start()
                cm.start()

            cw, cm = gather_copies(g, roww, rowm, semg)
            cw.wait()
            cm.wait()

            def row(i, carry):
                for c in range(D // L):
                    a = roww[i, pl.ds(c * L, L)]
                    b = rowm[i, pl.ds(c * L, L)]
                    roww[i, pl.ds(c * L, L)] = (a + b) * 0.5
                return carry

            lax.fori_loop(0, seq, row, 0, unroll=2)
            out_copy(g, roww, semo).start()

        buf_a = (roww_a, rowm_a, semg_a, semo_a)
        buf_b = (roww_b, rowm_b, semg_b, semo_b)

        cw, cm = gather_copies(0, roww_a, rowm_a, semg_a)
        cw.start()
        cm.start()

        def super_step(t, carry):
            step(2 * t, buf_a, buf_b)
            step(2 * t + 1, buf_b, buf_a)
            return carry

        lax.fori_loop(0, rows_w // 2, super_step, 0, unroll=False)
        out_copy(rows_w - 2, roww_a, semo_a).wait()
        out_copy(rows_w - 1, roww_b, semo_b).wait()

    return body(pid, mid, word_table, morph_table)


def kernel(phrase_ids, morph_ids, word_table, morph_table):
    return _fused_lookup(phrase_ids, morph_ids, word_table, morph_table)
